# baseline (speedup 1.0000x reference)
import functools

import jax
import jax.numpy as jnp
from jax import lax
from jax.experimental import pallas as pl
from jax.experimental.pallas import tpu as pltpu
from jax.experimental.pallas import tpu_sc as plsc

NC, NS, L = 2, 16, 16
NW = NC * NS
R, C = 128, 32768
RPW = R // NW

_mesh = plsc.VectorSubcoreMesh(core_axis_name="c", subcore_axis_name="s")


@functools.partial(
    pl.kernel,
    out_type=jax.ShapeDtypeStruct((R, C), jnp.float32),
    mesh=_mesh,
    scratch_types=[
        pltpu.VMEM((C,), jnp.float32),
        pltpu.VMEM((C,), jnp.float32),
        pltpu.VMEM((L,), jnp.float32),
    ],
)
def _sc_mask_rows(x_hbm, minfo_hbm, out_hbm, rowbuf, zbuf, mbuf):
    wid = lax.axis_index("s") * NC + lax.axis_index("c")
    pltpu.sync_copy(minfo_hbm.at[wid], mbuf)
    mvec = mbuf[...]

    def zfill(i, _):
        zbuf[pl.ds(i * L, L)] = jnp.zeros((L,), jnp.float32)
        return 0

    lax.fori_loop(0, C // L, zfill, 0)

    for j in range(RPW):
        mj = mvec[j]
        row = wid * RPW + j

        @pl.when(mj != 0.0)
        def _copy_row():
            pltpu.sync_copy(x_hbm.at[row], rowbuf)

            @pl.when(mj != 1.0)
            def _scale_row():
                def mul(i, _):
                    s = pl.ds(i * L, L)
                    rowbuf[s] = rowbuf[s] * mj
                    return 0

                lax.fori_loop(0, C // L, mul, 0)

            pltpu.sync_copy(rowbuf, out_hbm.at[row])

        @pl.when(mj == 0.0)
        def _zero_row():
            pltpu.sync_copy(zbuf, out_hbm.at[row])


def kernel(x, mask):
    minfo = jnp.zeros((NW, L), jnp.float32).at[:, :RPW].set(mask.reshape(NW, RPW))
    return _sc_mask_rows(x, minfo)
